# Initial kernel scaffold; baseline (speedup 1.0000x reference)
import jax
import jax.numpy as jnp
from jax.experimental import pallas as pl


def kernel(inputs, word_table, pos_table, W, b, gamma, beta, moving_mean, moving_var):
    raise NotImplementedError("write your pallas kernel here")


# speedup vs baseline: 1.9351x; 1.9351x over previous
import functools

import jax
import jax.numpy as jnp
from jax import lax
from jax.experimental import pallas as pl
from jax.experimental.pallas import tpu as pltpu
from jax.experimental.pallas import tpu_sc as plsc

BATCH = 1024
SEQ = 200
HIDDEN = 128
VOC = 100000
EPS = 1e-3

NC = 2
NS = 16
NW = NC * NS
ROWS = BATCH * SEQ
PER_W = ROWS // NW
CHUNK = 128
NCH = PER_W // CHUNK

TBLK = 1000
TGRID = VOC // TBLK


def _transform_body(tab_ref, w_ref, gamma_ref, var_ref, out_ref):
    scale = gamma_ref[...] * jax.lax.rsqrt(var_ref[...] + EPS)
    wp = w_ref[...] * scale
    out_ref[...] = jnp.dot(tab_ref[...], wp, preferred_element_type=jnp.float32)


def _cvec_body(word_ref, w_ref, gamma_ref, var_ref, b_ref, mean_ref, beta_ref, out_ref):
    scale = gamma_ref[...] * jax.lax.rsqrt(var_ref[...] + EPS)
    wp = w_ref[...] * scale
    bpp = scale * (b_ref[...] - mean_ref[...]) + beta_ref[...]
    out_ref[...] = jnp.dot(word_ref[...], wp, preferred_element_type=jnp.float32) + bpp


def _sc_gather(tp_hbm, idx_hbm, c_hbm, out_hbm, idx_v, buf, c_v, sem):
    wid = lax.axis_index("s") * NC + lax.axis_index("c")
    base = wid * PER_W
    pltpu.sync_copy(idx_hbm.at[pl.ds(base, PER_W)], idx_v)
    pltpu.sync_copy(c_hbm, c_v)

    def chunk_body(k, _):
        b0 = base + k * CHUNK
        pltpu.async_copy(tp_hbm.at[idx_v.at[pl.ds(k * CHUNK, CHUNK)]], buf, sem).wait()
        s0 = lax.rem(b0, SEQ)

        def row_body(i, _):
            s = lax.rem(s0 + i, SEQ)
            for j in range(HIDDEN // 16):
                sl = pl.ds(j * 16, 16)
                buf[i, sl] = buf[i, sl] + c_v[s, sl]
            return 0

        lax.fori_loop(0, CHUNK, row_body, 0, unroll=False)
        pltpu.sync_copy(buf, out_hbm.at[pl.ds(b0, CHUNK)])
        return 0

    lax.fori_loop(0, NCH, chunk_body, 0, unroll=False)


def kernel(inputs, word_table, pos_table, W, b, gamma, beta, moving_mean, moving_var):
    idx = inputs.reshape(-1).astype(jnp.int32)
    gamma2 = gamma.reshape(1, HIDDEN)
    var2 = moving_var.reshape(1, HIDDEN)
    b2 = b.reshape(1, HIDDEN)
    mean2 = moving_mean.reshape(1, HIDDEN)
    beta2 = beta.reshape(1, HIDDEN)

    tprime = pl.pallas_call(
        _transform_body,
        grid=(TGRID,),
        in_specs=[
            pl.BlockSpec((TBLK, HIDDEN), lambda i: (i, 0)),
            pl.BlockSpec((HIDDEN, HIDDEN), lambda i: (0, 0)),
            pl.BlockSpec((1, HIDDEN), lambda i: (0, 0)),
            pl.BlockSpec((1, HIDDEN), lambda i: (0, 0)),
        ],
        out_specs=pl.BlockSpec((TBLK, HIDDEN), lambda i: (i, 0)),
        out_shape=jax.ShapeDtypeStruct((VOC, HIDDEN), jnp.float32),
    )(pos_table, W, gamma2, var2)

    cvec = pl.pallas_call(
        _cvec_body,
        grid=(1,),
        in_specs=[
            pl.BlockSpec((SEQ, HIDDEN), lambda i: (0, 0)),
            pl.BlockSpec((HIDDEN, HIDDEN), lambda i: (0, 0)),
            pl.BlockSpec((1, HIDDEN), lambda i: (0, 0)),
            pl.BlockSpec((1, HIDDEN), lambda i: (0, 0)),
            pl.BlockSpec((1, HIDDEN), lambda i: (0, 0)),
            pl.BlockSpec((1, HIDDEN), lambda i: (0, 0)),
            pl.BlockSpec((1, HIDDEN), lambda i: (0, 0)),
        ],
        out_specs=pl.BlockSpec((SEQ, HIDDEN), lambda i: (0, 0)),
        out_shape=jax.ShapeDtypeStruct((SEQ, HIDDEN), jnp.float32),
    )(word_table, W, gamma2, var2, b2, mean2, beta2)

    sc_call = functools.partial(
        pl.kernel,
        out_type=jax.ShapeDtypeStruct((ROWS, HIDDEN), jnp.float32),
        mesh=plsc.VectorSubcoreMesh(core_axis_name="c", subcore_axis_name="s"),
        scratch_types=[
            pltpu.VMEM((PER_W,), jnp.int32),
            pltpu.VMEM((CHUNK, HIDDEN), jnp.float32),
            pltpu.VMEM((SEQ, HIDDEN), jnp.float32),
            pltpu.SemaphoreType.DMA,
        ],
    )(_sc_gather)
    out = sc_call(tprime, idx, cvec)
    return out.reshape(BATCH, SEQ, HIDDEN)


# speedup vs baseline: 2.7317x; 1.4116x over previous
import functools

import jax
import jax.numpy as jnp
from jax import lax
from jax.experimental import pallas as pl
from jax.experimental.pallas import tpu as pltpu
from jax.experimental.pallas import tpu_sc as plsc

BATCH = 1024
SEQ = 200
HIDDEN = 128
VOC = 100000
EPS = 1e-3

NC = 2
NS = 16
NW = NC * NS
ROWS = BATCH * SEQ
PER_W = ROWS // NW
CHUNK = 128
NCH = PER_W // CHUNK

TBLK = 2000
TGRID = VOC // TBLK


def _transform_body(tab_ref, w_ref, gamma_ref, var_ref, out_ref):
    scale = gamma_ref[...] * jax.lax.rsqrt(var_ref[...] + EPS)
    wp = w_ref[...] * scale
    out_ref[...] = jnp.dot(tab_ref[...], wp, preferred_element_type=jnp.float32)


def _cvec_body(word_ref, w_ref, gamma_ref, var_ref, b_ref, mean_ref, beta_ref, out_ref):
    scale = gamma_ref[...] * jax.lax.rsqrt(var_ref[...] + EPS)
    wp = w_ref[...] * scale
    bpp = scale * (b_ref[...] - mean_ref[...]) + beta_ref[...]
    out_ref[...] = jnp.dot(word_ref[...], wp, preferred_element_type=jnp.float32) + bpp


def _sc_gather(tp_hbm, idx_hbm, c_hbm, out_hbm, idx_v, c2_v, buf0, buf1,
               sg0, sg1, sw0, sw1):
    wid = lax.axis_index("s") * NC + lax.axis_index("c")
    base = wid * PER_W
    pltpu.sync_copy(idx_hbm.at[pl.ds(base, PER_W)], idx_v)
    pltpu.sync_copy(c_hbm, c2_v.at[pl.ds(0, SEQ)])
    pltpu.sync_copy(c_hbm, c2_v.at[pl.ds(SEQ, SEQ)])

    bufs = (buf0, buf1)
    sgs = (sg0, sg1)
    sws = (sw0, sw1)

    def gather_start(k, buf, sg):
        pltpu.async_copy(tp_hbm.at[idx_v.at[pl.ds(k * CHUNK, CHUNK)]], buf, sg)

    def gather_wait(k, buf, sg):
        pltpu.make_async_copy(
            tp_hbm.at[idx_v.at[pl.ds(k * CHUNK, CHUNK)]], buf, sg).wait()

    def write_start(k, buf, sw):
        pltpu.async_copy(buf, out_hbm.at[pl.ds(base + k * CHUNK, CHUNK)], sw)

    def write_wait(k, buf, sw):
        pltpu.make_async_copy(
            buf, out_hbm.at[pl.ds(base + k * CHUNK, CHUNK)], sw).wait()

    gather_start(0, buf0, sg0)

    def pair_body(i, _):
        for t in range(2):
            k = 2 * i + t
            buf, sg, sw = bufs[t], sgs[t], sws[t]
            nbuf, nsg, nsw = bufs[1 - t], sgs[1 - t], sws[1 - t]

            @pl.when(k + 1 < NCH)
            def _():
                @pl.when(k >= 1)
                def _():
                    write_wait(k - 1, nbuf, nsw)
                gather_start(k + 1, nbuf, nsg)

            gather_wait(k, buf, sg)
            s0 = lax.rem(k * CHUNK, SEQ)

            def row_body(idx_i, _):
                c_row = s0 + idx_i
                for j in range(HIDDEN // 16):
                    sl = pl.ds(j * 16, 16)
                    plsc.addupdate(buf.at[idx_i, sl], c2_v[c_row, sl])
                return 0

            lax.fori_loop(0, CHUNK, row_body, 0, unroll=4)
            write_start(k, buf, sw)
        return 0

    lax.fori_loop(0, NCH // 2, pair_body, 0, unroll=False)
    write_wait(NCH - 2, buf0, sw0)
    write_wait(NCH - 1, buf1, sw1)


def kernel(inputs, word_table, pos_table, W, b, gamma, beta, moving_mean, moving_var):
    idx = inputs.reshape(-1).astype(jnp.int32)
    gamma2 = gamma.reshape(1, HIDDEN)
    var2 = moving_var.reshape(1, HIDDEN)
    b2 = b.reshape(1, HIDDEN)
    mean2 = moving_mean.reshape(1, HIDDEN)
    beta2 = beta.reshape(1, HIDDEN)

    tprime = pl.pallas_call(
        _transform_body,
        grid=(TGRID,),
        in_specs=[
            pl.BlockSpec((TBLK, HIDDEN), lambda i: (i, 0)),
            pl.BlockSpec((HIDDEN, HIDDEN), lambda i: (0, 0)),
            pl.BlockSpec((1, HIDDEN), lambda i: (0, 0)),
            pl.BlockSpec((1, HIDDEN), lambda i: (0, 0)),
        ],
        out_specs=pl.BlockSpec((TBLK, HIDDEN), lambda i: (i, 0)),
        out_shape=jax.ShapeDtypeStruct((VOC, HIDDEN), jnp.float32),
    )(pos_table, W, gamma2, var2)

    cvec = pl.pallas_call(
        _cvec_body,
        grid=(1,),
        in_specs=[
            pl.BlockSpec((SEQ, HIDDEN), lambda i: (0, 0)),
            pl.BlockSpec((HIDDEN, HIDDEN), lambda i: (0, 0)),
            pl.BlockSpec((1, HIDDEN), lambda i: (0, 0)),
            pl.BlockSpec((1, HIDDEN), lambda i: (0, 0)),
            pl.BlockSpec((1, HIDDEN), lambda i: (0, 0)),
            pl.BlockSpec((1, HIDDEN), lambda i: (0, 0)),
            pl.BlockSpec((1, HIDDEN), lambda i: (0, 0)),
        ],
        out_specs=pl.BlockSpec((SEQ, HIDDEN), lambda i: (0, 0)),
        out_shape=jax.ShapeDtypeStruct((SEQ, HIDDEN), jnp.float32),
    )(word_table, W, gamma2, var2, b2, mean2, beta2)

    sc_call = functools.partial(
        pl.kernel,
        out_type=jax.ShapeDtypeStruct((ROWS, HIDDEN), jnp.float32),
        mesh=plsc.VectorSubcoreMesh(core_axis_name="c", subcore_axis_name="s"),
        scratch_types=[
            pltpu.VMEM((PER_W,), jnp.int32),
            pltpu.VMEM((2 * SEQ, HIDDEN), jnp.float32),
            pltpu.VMEM((CHUNK, HIDDEN), jnp.float32),
            pltpu.VMEM((CHUNK, HIDDEN), jnp.float32),
            pltpu.SemaphoreType.DMA,
            pltpu.SemaphoreType.DMA,
            pltpu.SemaphoreType.DMA,
            pltpu.SemaphoreType.DMA,
        ],
    )(_sc_gather)
    out = sc_call(tprime, idx, cvec)
    return out.reshape(BATCH, SEQ, HIDDEN)


# speedup vs baseline: 3.0699x; 1.1238x over previous
import functools

import jax
import jax.numpy as jnp
from jax import lax
from jax.experimental import pallas as pl
from jax.experimental.pallas import tpu as pltpu
from jax.experimental.pallas import tpu_sc as plsc

BATCH = 1024
SEQ = 200
HIDDEN = 128
VOC = 100000
EPS = 1e-3

NC = 2
NS = 16
NW = NC * NS
ROWS = BATCH * SEQ
PER_W = ROWS // NW
CHUNK = 128
NCH = PER_W // CHUNK

TBLK = 2000
TGRID = VOC // TBLK


def _transform_body(tab_ref, w_ref, gamma_ref, var_ref, out_ref):
    scale = gamma_ref[...] * jax.lax.rsqrt(var_ref[...] + EPS)
    wp = w_ref[...] * scale
    out_ref[...] = jnp.dot(tab_ref[...], wp, preferred_element_type=jnp.float32)


def _cvec_body(word_ref, w_ref, gamma_ref, var_ref, b_ref, mean_ref, beta_ref, out_ref):
    scale = gamma_ref[...] * jax.lax.rsqrt(var_ref[...] + EPS)
    wp = w_ref[...] * scale
    bpp = scale * (b_ref[...] - mean_ref[...]) + beta_ref[...]
    out_ref[...] = jnp.dot(word_ref[...], wp, preferred_element_type=jnp.float32) + bpp


NBUF = 3
NMAIN = ((NCH - 1) // NBUF) * NBUF


def _sc_gather(tp_hbm, idx_hbm, c_hbm, out_hbm, idx_v, c2_v, buf0, buf1, buf2,
               sg0, sg1, sg2, sw0, sw1, sw2):
    wid = lax.axis_index("s") * NC + lax.axis_index("c")
    base = wid * PER_W
    pltpu.sync_copy(idx_hbm.at[pl.ds(base, PER_W)], idx_v)
    pltpu.sync_copy(c_hbm, c2_v.at[pl.ds(0, SEQ)])
    pltpu.sync_copy(c_hbm, c2_v.at[pl.ds(SEQ, SEQ)])

    bufs = (buf0, buf1, buf2)
    sgs = (sg0, sg1, sg2)
    sws = (sw0, sw1, sw2)

    def gather_start(k, r):
        pltpu.async_copy(
            tp_hbm.at[idx_v.at[pl.ds(k * CHUNK, CHUNK)]], bufs[r], sgs[r])

    def gather_wait(k, r):
        pltpu.make_async_copy(
            tp_hbm.at[idx_v.at[pl.ds(k * CHUNK, CHUNK)]], bufs[r], sgs[r]).wait()

    def write_start(k, r):
        pltpu.async_copy(bufs[r], out_hbm.at[pl.ds(base + k * CHUNK, CHUNK)], sws[r])

    def write_wait(k, r):
        pltpu.make_async_copy(
            bufs[r], out_hbm.at[pl.ds(base + k * CHUNK, CHUNK)], sws[r]).wait()

    def add_c(k, r):
        buf = bufs[r]
        s0 = lax.rem(k * CHUNK, SEQ)

        def row_body(i, _):
            c_row = s0 + i
            for j in range(HIDDEN // 16):
                sl = pl.ds(j * 16, 16)
                plsc.addupdate(buf.at[i, sl], c2_v[c_row, sl])
            return 0

        lax.fori_loop(0, CHUNK, row_body, 0, unroll=4)

    gather_start(0, 0)

    def ring_body(i, _):
        for t in range(NBUF):
            k = NBUF * i + t
            r = t

            nr = (r + 1) % NBUF

            @pl.when(k >= NBUF - 1)
            def _():
                write_wait(k - (NBUF - 1), nr)

            gather_start(k + 1, nr)
            gather_wait(k, r)
            add_c(k, r)
            write_start(k, r)
        return 0

    lax.fori_loop(0, NMAIN // NBUF, ring_body, 0, unroll=False)
    for k in range(NMAIN, NCH):
        r = k % NBUF
        if k + 1 < NCH:
            nr = (r + 1) % NBUF
            write_wait(k + 1 - NBUF, nr)
            gather_start(k + 1, nr)
        gather_wait(k, r)
        add_c(k, r)
        write_start(k, r)
    for k in range(NCH - NBUF, NCH):
        write_wait(k, k % NBUF)


def kernel(inputs, word_table, pos_table, W, b, gamma, beta, moving_mean, moving_var):
    idx = inputs.reshape(-1).astype(jnp.int32)
    gamma2 = gamma.reshape(1, HIDDEN)
    var2 = moving_var.reshape(1, HIDDEN)
    b2 = b.reshape(1, HIDDEN)
    mean2 = moving_mean.reshape(1, HIDDEN)
    beta2 = beta.reshape(1, HIDDEN)

    tprime = pl.pallas_call(
        _transform_body,
        grid=(TGRID,),
        in_specs=[
            pl.BlockSpec((TBLK, HIDDEN), lambda i: (i, 0)),
            pl.BlockSpec((HIDDEN, HIDDEN), lambda i: (0, 0)),
            pl.BlockSpec((1, HIDDEN), lambda i: (0, 0)),
            pl.BlockSpec((1, HIDDEN), lambda i: (0, 0)),
        ],
        out_specs=pl.BlockSpec((TBLK, HIDDEN), lambda i: (i, 0)),
        out_shape=jax.ShapeDtypeStruct((VOC, HIDDEN), jnp.float32),
    )(pos_table, W, gamma2, var2)

    cvec = pl.pallas_call(
        _cvec_body,
        grid=(1,),
        in_specs=[
            pl.BlockSpec((SEQ, HIDDEN), lambda i: (0, 0)),
            pl.BlockSpec((HIDDEN, HIDDEN), lambda i: (0, 0)),
            pl.BlockSpec((1, HIDDEN), lambda i: (0, 0)),
            pl.BlockSpec((1, HIDDEN), lambda i: (0, 0)),
            pl.BlockSpec((1, HIDDEN), lambda i: (0, 0)),
            pl.BlockSpec((1, HIDDEN), lambda i: (0, 0)),
            pl.BlockSpec((1, HIDDEN), lambda i: (0, 0)),
        ],
        out_specs=pl.BlockSpec((SEQ, HIDDEN), lambda i: (0, 0)),
        out_shape=jax.ShapeDtypeStruct((SEQ, HIDDEN), jnp.float32),
    )(word_table, W, gamma2, var2, b2, mean2, beta2)

    sc_call = functools.partial(
        pl.kernel,
        out_type=jax.ShapeDtypeStruct((ROWS, HIDDEN), jnp.float32),
        mesh=plsc.VectorSubcoreMesh(core_axis_name="c", subcore_axis_name="s"),
        scratch_types=(
            [pltpu.VMEM((PER_W,), jnp.int32),
             pltpu.VMEM((2 * SEQ, HIDDEN), jnp.float32)]
            + [pltpu.VMEM((CHUNK, HIDDEN), jnp.float32)] * NBUF
            + [pltpu.SemaphoreType.DMA] * (2 * NBUF)
        ),
    )(_sc_gather)
    out = sc_call(tprime, idx, cvec)
    return out.reshape(BATCH, SEQ, HIDDEN)


# speedup vs baseline: 5.5386x; 1.8042x over previous
import functools

import jax
import jax.numpy as jnp
from jax import lax
from jax.experimental import pallas as pl
from jax.experimental.pallas import tpu as pltpu
from jax.experimental.pallas import tpu_sc as plsc

BATCH = 1024
SEQ = 200
HIDDEN = 128
VOC = 100000
EPS = 1e-3

NC = 2
NS = 16
NW = NC * NS
ROWS = BATCH * SEQ
PER_W = ROWS // NW
CHUNK = 128
NCH = PER_W // CHUNK

TBLK = 2000
TGRID = VOC // TBLK


def _transform_body(tab_ref, w_ref, gamma_ref, var_ref, out_ref):
    scale = gamma_ref[...] * jax.lax.rsqrt(var_ref[...] + EPS)
    wp = w_ref[...] * scale
    out_ref[...] = jnp.dot(tab_ref[...], wp, preferred_element_type=jnp.float32)


def _cvec_body(word_ref, w_ref, gamma_ref, var_ref, b_ref, mean_ref, beta_ref, out_ref):
    scale = gamma_ref[...] * jax.lax.rsqrt(var_ref[...] + EPS)
    wp = w_ref[...] * scale
    bpp = scale * (b_ref[...] - mean_ref[...]) + beta_ref[...]
    out_ref[...] = jnp.dot(word_ref[...], wp, preferred_element_type=jnp.float32) + bpp


NBUF = 4
LOOK = 2
NMAIN = ((NCH - LOOK) // NBUF) * NBUF
C2ROWS = 320


def _sc_gather(tp_hbm, idx_hbm, c_hbm, out_hbm, idx_v, c2_v,
               buf0, buf1, buf2, buf3,
               sg0, sg1, sg2, sg3, sw0, sw1, sw2, sw3):
    wid = lax.axis_index("s") * NC + lax.axis_index("c")
    base = wid * PER_W
    pltpu.sync_copy(idx_hbm.at[pl.ds(base, PER_W)], idx_v)
    pltpu.sync_copy(c_hbm, c2_v.at[pl.ds(0, SEQ)])
    pltpu.sync_copy(c_hbm.at[pl.ds(0, C2ROWS - SEQ)], c2_v.at[pl.ds(SEQ, C2ROWS - SEQ)])

    bufs = (buf0, buf1, buf2, buf3)
    sgs = (sg0, sg1, sg2, sg3)
    sws = (sw0, sw1, sw2, sw3)

    def gather_start(k, r):
        pltpu.async_copy(
            tp_hbm.at[idx_v.at[pl.ds(k * CHUNK, CHUNK)]], bufs[r], sgs[r])

    def gather_wait(k, r):
        pltpu.make_async_copy(
            tp_hbm.at[idx_v.at[pl.ds(k * CHUNK, CHUNK)]], bufs[r], sgs[r]).wait()

    def write_start(k, r):
        pltpu.async_copy(bufs[r], out_hbm.at[pl.ds(base + k * CHUNK, CHUNK)], sws[r])

    def write_wait(k, r):
        pltpu.make_async_copy(
            bufs[r], out_hbm.at[pl.ds(base + k * CHUNK, CHUNK)], sws[r]).wait()

    def add_c(k, r):
        buf = bufs[r]
        s0 = lax.rem(k * CHUNK, SEQ)

        def row_body(i):
            c_row = s0 + i
            for j in range(HIDDEN // 16):
                sl = pl.ds(j * 16, 16)
                plsc.addupdate(buf.at[i, sl], c2_v[c_row, sl])

        plsc.parallel_loop(0, CHUNK, 1, unroll=4)(row_body)

    for p in range(LOOK):
        gather_start(p, p)

    def ring_body(i, _):
        for t in range(NBUF):
            k = NBUF * i + t
            r = t
            nr = (t + LOOK) % NBUF

            @pl.when(k + LOOK >= NBUF)
            def _():
                write_wait(k + LOOK - NBUF, nr)

            gather_start(k + LOOK, nr)
            gather_wait(k, r)
            add_c(k, r)
            write_start(k, r)
        return 0

    lax.fori_loop(0, NMAIN // NBUF, ring_body, 0, unroll=False)
    for k in range(NMAIN, NCH):
        r = k % NBUF
        if k + LOOK < NCH:
            nr = (k + LOOK) % NBUF
            write_wait(k + LOOK - NBUF, nr)
            gather_start(k + LOOK, nr)
        gather_wait(k, r)
        add_c(k, r)
        write_start(k, r)
    for k in range(NCH - NBUF, NCH):
        write_wait(k, k % NBUF)


def kernel(inputs, word_table, pos_table, W, b, gamma, beta, moving_mean, moving_var):
    idx = inputs.reshape(-1).astype(jnp.int32)
    gamma2 = gamma.reshape(1, HIDDEN)
    var2 = moving_var.reshape(1, HIDDEN)
    b2 = b.reshape(1, HIDDEN)
    mean2 = moving_mean.reshape(1, HIDDEN)
    beta2 = beta.reshape(1, HIDDEN)

    tprime = pl.pallas_call(
        _transform_body,
        grid=(TGRID,),
        in_specs=[
            pl.BlockSpec((TBLK, HIDDEN), lambda i: (i, 0)),
            pl.BlockSpec((HIDDEN, HIDDEN), lambda i: (0, 0)),
            pl.BlockSpec((1, HIDDEN), lambda i: (0, 0)),
            pl.BlockSpec((1, HIDDEN), lambda i: (0, 0)),
        ],
        out_specs=pl.BlockSpec((TBLK, HIDDEN), lambda i: (i, 0)),
        out_shape=jax.ShapeDtypeStruct((VOC, HIDDEN), jnp.float32),
    )(pos_table, W, gamma2, var2)

    cvec = pl.pallas_call(
        _cvec_body,
        grid=(1,),
        in_specs=[
            pl.BlockSpec((SEQ, HIDDEN), lambda i: (0, 0)),
            pl.BlockSpec((HIDDEN, HIDDEN), lambda i: (0, 0)),
            pl.BlockSpec((1, HIDDEN), lambda i: (0, 0)),
            pl.BlockSpec((1, HIDDEN), lambda i: (0, 0)),
            pl.BlockSpec((1, HIDDEN), lambda i: (0, 0)),
            pl.BlockSpec((1, HIDDEN), lambda i: (0, 0)),
            pl.BlockSpec((1, HIDDEN), lambda i: (0, 0)),
        ],
        out_specs=pl.BlockSpec((SEQ, HIDDEN), lambda i: (0, 0)),
        out_shape=jax.ShapeDtypeStruct((SEQ, HIDDEN), jnp.float32),
    )(word_table, W, gamma2, var2, b2, mean2, beta2)

    sc_call = functools.partial(
        pl.kernel,
        out_type=jax.ShapeDtypeStruct((ROWS, HIDDEN), jnp.float32),
        mesh=plsc.VectorSubcoreMesh(core_axis_name="c", subcore_axis_name="s"),
        scratch_types=(
            [pltpu.VMEM((PER_W,), jnp.int32),
             pltpu.VMEM((2 * SEQ, HIDDEN), jnp.float32)]
            + [pltpu.VMEM((CHUNK, HIDDEN), jnp.float32)] * NBUF
            + [pltpu.SemaphoreType.DMA] * (2 * NBUF)
        ),
    )(_sc_gather)
    out = sc_call(tprime, idx, cvec)
    return out.reshape(BATCH, SEQ, HIDDEN)


# speedup vs baseline: 5.5630x; 1.0044x over previous
import functools

import jax
import jax.numpy as jnp
from jax import lax
from jax.experimental import pallas as pl
from jax.experimental.pallas import tpu as pltpu
from jax.experimental.pallas import tpu_sc as plsc

BATCH = 1024
SEQ = 200
HIDDEN = 128
VOC = 100000
EPS = 1e-3

NC = 2
NS = 16
NW = NC * NS
ROWS = BATCH * SEQ
PER_W = ROWS // NW
CHUNK = 128
NCH = PER_W // CHUNK

TBLK = 2000
TGRID = VOC // TBLK


def _transform_body(tab_ref, word_ref, w_ref, gamma_ref, var_ref, b_ref,
                    mean_ref, beta_ref, out_ref, c_ref):
    scale = gamma_ref[...] * jax.lax.rsqrt(var_ref[...] + EPS)
    wp = w_ref[...] * scale
    out_ref[...] = jnp.dot(tab_ref[...], wp, preferred_element_type=jnp.float32)
    bpp = scale * (b_ref[...] - mean_ref[...]) + beta_ref[...]
    c_ref[...] = jnp.dot(word_ref[...], wp, preferred_element_type=jnp.float32) + bpp


NBUF = 4
LOOK = 2
NMAIN = ((NCH - LOOK) // NBUF) * NBUF
C2ROWS = 320


def _sc_gather(tp_hbm, idx_hbm, c_hbm, out_hbm, idx_v, c2_v,
               buf0, buf1, buf2, buf3,
               sg0, sg1, sg2, sg3, sw0, sw1, sw2, sw3):
    wid = lax.axis_index("s") * NC + lax.axis_index("c")
    base = wid * PER_W
    pltpu.sync_copy(idx_hbm.at[pl.ds(base, PER_W)], idx_v)
    pltpu.sync_copy(c_hbm, c2_v.at[pl.ds(0, SEQ)])
    pltpu.sync_copy(c_hbm.at[pl.ds(0, C2ROWS - SEQ)], c2_v.at[pl.ds(SEQ, C2ROWS - SEQ)])

    bufs = (buf0, buf1, buf2, buf3)
    sgs = (sg0, sg1, sg2, sg3)
    sws = (sw0, sw1, sw2, sw3)

    def gather_start(k, r):
        pltpu.async_copy(
            tp_hbm.at[idx_v.at[pl.ds(k * CHUNK, CHUNK)]], bufs[r], sgs[r])

    def gather_wait(k, r):
        pltpu.make_async_copy(
            tp_hbm.at[idx_v.at[pl.ds(k * CHUNK, CHUNK)]], bufs[r], sgs[r]).wait()

    def write_start(k, r):
        pltpu.async_copy(bufs[r], out_hbm.at[pl.ds(base + k * CHUNK, CHUNK)], sws[r])

    def write_wait(k, r):
        pltpu.make_async_copy(
            bufs[r], out_hbm.at[pl.ds(base + k * CHUNK, CHUNK)], sws[r]).wait()

    def add_c(k, r):
        buf = bufs[r]
        s0 = lax.rem(k * CHUNK, SEQ)

        def row_body(i):
            c_row = s0 + i
            for j in range(HIDDEN // 16):
                sl = pl.ds(j * 16, 16)
                plsc.addupdate(buf.at[i, sl], c2_v[c_row, sl])

        plsc.parallel_loop(0, CHUNK, 1, unroll=4)(row_body)

    for p in range(LOOK):
        gather_start(p, p)

    def ring_body(i, _):
        for t in range(NBUF):
            k = NBUF * i + t
            r = t
            nr = (t + LOOK) % NBUF

            @pl.when(k + LOOK >= NBUF)
            def _():
                write_wait(k + LOOK - NBUF, nr)

            gather_start(k + LOOK, nr)
            gather_wait(k, r)
            add_c(k, r)
            write_start(k, r)
        return 0

    lax.fori_loop(0, NMAIN // NBUF, ring_body, 0, unroll=False)
    for k in range(NMAIN, NCH):
        r = k % NBUF
        if k + LOOK < NCH:
            nr = (k + LOOK) % NBUF
            write_wait(k + LOOK - NBUF, nr)
            gather_start(k + LOOK, nr)
        gather_wait(k, r)
        add_c(k, r)
        write_start(k, r)
    for k in range(NCH - NBUF, NCH):
        write_wait(k, k % NBUF)


def kernel(inputs, word_table, pos_table, W, b, gamma, beta, moving_mean, moving_var):
    idx = inputs.reshape(-1).astype(jnp.int32)
    gamma2 = gamma.reshape(1, HIDDEN)
    var2 = moving_var.reshape(1, HIDDEN)
    b2 = b.reshape(1, HIDDEN)
    mean2 = moving_mean.reshape(1, HIDDEN)
    beta2 = beta.reshape(1, HIDDEN)

    tprime, cvec = pl.pallas_call(
        _transform_body,
        grid=(TGRID,),
        in_specs=[
            pl.BlockSpec((TBLK, HIDDEN), lambda i: (i, 0)),
            pl.BlockSpec((SEQ, HIDDEN), lambda i: (0, 0)),
            pl.BlockSpec((HIDDEN, HIDDEN), lambda i: (0, 0)),
            pl.BlockSpec((1, HIDDEN), lambda i: (0, 0)),
            pl.BlockSpec((1, HIDDEN), lambda i: (0, 0)),
            pl.BlockSpec((1, HIDDEN), lambda i: (0, 0)),
            pl.BlockSpec((1, HIDDEN), lambda i: (0, 0)),
            pl.BlockSpec((1, HIDDEN), lambda i: (0, 0)),
        ],
        out_specs=[
            pl.BlockSpec((TBLK, HIDDEN), lambda i: (i, 0)),
            pl.BlockSpec((SEQ, HIDDEN), lambda i: (0, 0)),
        ],
        out_shape=[
            jax.ShapeDtypeStruct((VOC, HIDDEN), jnp.float32),
            jax.ShapeDtypeStruct((SEQ, HIDDEN), jnp.float32),
        ],
    )(pos_table, word_table, W, gamma2, var2, b2, mean2, beta2)

    sc_call = functools.partial(
        pl.kernel,
        out_type=jax.ShapeDtypeStruct((ROWS, HIDDEN), jnp.float32),
        mesh=plsc.VectorSubcoreMesh(core_axis_name="c", subcore_axis_name="s"),
        scratch_types=(
            [pltpu.VMEM((PER_W,), jnp.int32),
             pltpu.VMEM((2 * SEQ, HIDDEN), jnp.float32)]
            + [pltpu.VMEM((CHUNK, HIDDEN), jnp.float32)] * NBUF
            + [pltpu.SemaphoreType.DMA] * (2 * NBUF)
        ),
    )(_sc_gather)
    out = sc_call(tprime, idx, cvec)
    return out.reshape(BATCH, SEQ, HIDDEN)


# speedup vs baseline: 5.5764x; 1.0024x over previous
import functools

import jax
import jax.numpy as jnp
from jax import lax
from jax.experimental import pallas as pl
from jax.experimental.pallas import tpu as pltpu
from jax.experimental.pallas import tpu_sc as plsc

BATCH = 1024
SEQ = 200
HIDDEN = 128
VOC = 100000
EPS = 1e-3

NC = 2
NS = 16
NW = NC * NS
ROWS = BATCH * SEQ
PER_W = ROWS // NW
CHUNK = 128
NCH = PER_W // CHUNK

TBLK = 2000
TGRID = VOC // TBLK

NBUF = 4
LOOK = 2
NMAIN = ((NCH - LOOK) // NBUF) * NBUF
C2ROWS = 320


def _transform_body(tab_ref, word_ref, w_ref, gamma_ref, var_ref, b_ref,
                    mean_ref, beta_ref, out_ref, c_ref):
    scale = gamma_ref[...] * jax.lax.rsqrt(var_ref[...] + EPS)
    wp = w_ref[...] * scale
    out_ref[...] = jnp.dot(tab_ref[...], wp, preferred_element_type=jnp.float32)
    bpp = scale * (b_ref[...] - mean_ref[...]) + beta_ref[...]
    c_ref[...] = jnp.dot(word_ref[...], wp, preferred_element_type=jnp.float32) + bpp


def _sc_gather(tp_hbm, idx_hbm, c_hbm, out_hbm, idx_v, c2_v,
               buf0, buf1, buf2, buf3,
               sg0, sg1, sg2, sg3, sw0, sw1, sw2, sw3):
    wid = lax.axis_index("s") * NC + lax.axis_index("c")
    base = wid * PER_W
    pltpu.sync_copy(idx_hbm.at[pl.ds(base, PER_W)], idx_v)
    pltpu.sync_copy(c_hbm, c2_v.at[pl.ds(0, SEQ)])
    pltpu.sync_copy(c_hbm.at[pl.ds(0, C2ROWS - SEQ)], c2_v.at[pl.ds(SEQ, C2ROWS - SEQ)])

    bufs = (buf0, buf1, buf2, buf3)
    sgs = (sg0, sg1, sg2, sg3)
    sws = (sw0, sw1, sw2, sw3)

    def gather_start(k, r):
        pltpu.async_copy(
            tp_hbm.at[idx_v.at[pl.ds(k * CHUNK, CHUNK)]], bufs[r], sgs[r])

    def gather_wait(k, r):
        pltpu.make_async_copy(
            tp_hbm.at[idx_v.at[pl.ds(k * CHUNK, CHUNK)]], bufs[r], sgs[r]).wait()

    def write_start(k, r):
        pltpu.async_copy(bufs[r], out_hbm.at[pl.ds(base + k * CHUNK, CHUNK)], sws[r])

    def write_wait(k, r):
        pltpu.make_async_copy(
            bufs[r], out_hbm.at[pl.ds(base + k * CHUNK, CHUNK)], sws[r]).wait()

    def add_c(k, r):
        buf = bufs[r]
        s0 = lax.rem(k * CHUNK, SEQ)

        def row_body(i):
            c_row = s0 + i
            for j in range(HIDDEN // 16):
                sl = pl.ds(j * 16, 16)
                plsc.addupdate(buf.at[i, sl], c2_v[c_row, sl])

        plsc.parallel_loop(0, CHUNK, 1, unroll=4)(row_body)

    for p in range(LOOK):
        gather_start(p, p)

    def ring_body(i, _):
        for t in range(NBUF):
            k = NBUF * i + t
            r = t
            nr = (t + LOOK) % NBUF

            @pl.when(k + LOOK >= NBUF)
            def _():
                write_wait(k + LOOK - NBUF, nr)

            gather_start(k + LOOK, nr)
            gather_wait(k, r)
            add_c(k, r)
            write_start(k, r)
        return 0

    lax.fori_loop(0, NMAIN // NBUF, ring_body, 0, unroll=False)
    for k in range(NMAIN, NCH):
        r = k % NBUF
        if k + LOOK < NCH:
            nr = (k + LOOK) % NBUF
            write_wait(k + LOOK - NBUF, nr)
            gather_start(k + LOOK, nr)
        gather_wait(k, r)
        add_c(k, r)
        write_start(k, r)
    for k in range(NCH - NBUF, NCH):
        write_wait(k, k % NBUF)


def kernel(inputs, word_table, pos_table, W, b, gamma, beta, moving_mean, moving_var):
    idx = inputs.reshape(-1).astype(jnp.int32)
    gamma2 = gamma.reshape(1, HIDDEN)
    var2 = moving_var.reshape(1, HIDDEN)
    b2 = b.reshape(1, HIDDEN)
    mean2 = moving_mean.reshape(1, HIDDEN)
    beta2 = beta.reshape(1, HIDDEN)

    full = lambda shp: pl.BlockSpec(shp, lambda i: (0, 0))
    tprime, cvec = pl.pallas_call(
        _transform_body,
        grid=(TGRID,),
        in_specs=[
            pl.BlockSpec((TBLK, HIDDEN), lambda i: (i, 0)),
            full((SEQ, HIDDEN)),
            full((HIDDEN, HIDDEN)),
            full((1, HIDDEN)),
            full((1, HIDDEN)),
            full((1, HIDDEN)),
            full((1, HIDDEN)),
            full((1, HIDDEN)),
        ],
        out_specs=[
            pl.BlockSpec((TBLK, HIDDEN), lambda i: (i, 0)),
            full((SEQ, HIDDEN)),
        ],
        out_shape=[
            jax.ShapeDtypeStruct((VOC, HIDDEN), jnp.float32),
            jax.ShapeDtypeStruct((SEQ, HIDDEN), jnp.float32),
        ],
    )(pos_table, word_table, W, gamma2, var2, b2, mean2, beta2)

    sc_call = functools.partial(
        pl.kernel,
        out_type=jax.ShapeDtypeStruct((ROWS, HIDDEN), jnp.float32),
        mesh=plsc.VectorSubcoreMesh(core_axis_name="c", subcore_axis_name="s"),
        scratch_types=(
            [pltpu.VMEM((PER_W,), jnp.int32),
             pltpu.VMEM((C2ROWS, HIDDEN), jnp.float32)]
            + [pltpu.VMEM((CHUNK, HIDDEN), jnp.float32)] * NBUF
            + [pltpu.SemaphoreType.DMA] * (2 * NBUF)
        ),
    )(_sc_gather)
    out = sc_call(tprime, idx, cvec)
    return out.reshape(BATCH, SEQ, HIDDEN)
